# baseline (speedup 1.0000x reference)
import functools

import jax
import jax.numpy as jnp
from jax import lax
from jax.experimental import pallas as pl
from jax.experimental.pallas import tpu as pltpu
from jax.experimental.pallas import tpu_sc as plsc

N = 10000
E = 320000
NC = 2
NS = 16
NW = NC * NS
EPW = 10240
EP = NW * EPW
ACCN = 10240
RPT = ACCN // NS
ROUT = 400
ZR = 64

_EPS = 1e-5


@functools.lru_cache(maxsize=None)
def _make_agg(C):
    if C == 128:
        CH, R, G = 64, 4, 2
    else:
        CH, R, G = 128, 4, 2
    K = R - G
    NCHUNK = EPW // CH
    zr = 32
    mesh = plsc.VectorSubcoreMesh(
        core_axis_name="c", subcore_axis_name="s", num_cores=NC, num_subcores=NS
    )

    @functools.partial(
        pl.kernel,
        out_type=jax.ShapeDtypeStruct((NC, N, C), jnp.float32),
        mesh=mesh,
        compiler_params=pltpu.CompilerParams(use_tc_tiling_on_sc=False),
        scratch_types=[
            pltpu.VMEM_SHARED((ACCN, C), jnp.float32),
            [pltpu.VMEM((CH,), jnp.int32) for _ in range(R)],
            [pltpu.VMEM((CH,), jnp.int32) for _ in range(2 * R)],
            [pltpu.VMEM((CH, C), jnp.float32) for _ in range(R)],
            pltpu.VMEM((zr, C), jnp.float32),
            [pltpu.SemaphoreType.DMA for _ in range(R)],
            [pltpu.SemaphoreType.DMA for _ in range(2 * R)],
            [pltpu.SemaphoreType.DMA for _ in range(R)],
            [pltpu.SemaphoreType.DMA for _ in range(R)],
        ],
    )
    def agg(h_hbm, src_hbm, dst_hbm, out_hbm, acc, sbs, dbs, rbs, zb,
            sss, iss, gss, css):
        c = lax.axis_index("c")
        s = lax.axis_index("s")
        wid = c * NS + s

        def zrow(i, _):
            def zcol(k, _):
                zb[i, pl.ds(k * 16, 16)] = jnp.zeros((16,), jnp.float32)
                return 0
            return lax.fori_loop(0, C // 16, zcol, 0)
        lax.fori_loop(0, zr, zrow, 0)
        start = pl.multiple_of(s * RPT, 8)
        for r in range(RPT // zr):
            pltpu.sync_copy(zb, acc.at[pl.ds(start + r * zr, zr)])
        plsc.subcore_barrier()

        def src_cp(j, v):
            off = pl.multiple_of(wid * EPW + j * CH, 8)
            return pltpu.make_async_copy(src_hbm.at[pl.ds(off, CH)],
                                         sbs[v % R], sss[v % R])

        def dst_cp(j, v):
            off = pl.multiple_of(wid * EPW + j * CH, 8)
            return pltpu.make_async_copy(dst_hbm.at[pl.ds(off, CH)],
                                         dbs[v % (2 * R)], iss[v % (2 * R)])

        def row_cp(j, v):
            return pltpu.make_async_copy(h_hbm.at[sbs[v % R]],
                                         rbs[v % R], gss[v % R])

        def add_start(v):
            pltpu.async_copy(rbs[v % R], acc.at[dbs[v % (2 * R)]],
                             css[v % R], add=True)

        def add_wait(v):
            pltpu.make_async_copy(rbs[v % R], acc.at[dbs[v % (2 * R)]],
                                  css[v % R]).wait()

        for q in range(R):
            src_cp(q, q).start()
            dst_cp(q, q).start()
        for q in range(G):
            src_cp(q, q).wait()
            row_cp(q, q).start()

        def step(j, v):
            row_cp(j, v).wait()

            @pl.when(j + R < NCHUNK)
            def _():
                src_cp(j + R, v + R).start()
                dst_cp(j + R, v + R).start()

            dst_cp(j, v).wait()
            add_start(v)

            @pl.when(j >= K)
            def _():
                add_wait(v - K)

            @pl.when(j + G < NCHUNK)
            def _():
                src_cp(j + G, v + G).wait()
                row_cp(j + G, v + G).start()

        def outer(jo, _):
            for v in range(2 * R):
                step(jo * 2 * R + v, v)
            return 0

        lax.fori_loop(0, NCHUNK // (2 * R), outer, 0)

        for k in range(K):
            add_wait(NCHUNK - K + k)
        plsc.subcore_barrier()

        @pl.when(s < NS - 1)
        def _():
            pltpu.sync_copy(acc.at[pl.ds(start, RPT)],
                            out_hbm.at[c, pl.ds(start, RPT)])

        @pl.when(s == NS - 1)
        def _():
            pltpu.sync_copy(acc.at[pl.ds(start, ROUT)],
                            out_hbm.at[c, pl.ds(start, ROUT)])

    return agg


def _agg(h, srcr, dstr):
    return _make_agg(h.shape[1])(h, srcr, dstr)


def _norm(t):
    m = jnp.mean(t, axis=0, keepdims=True)
    v = jnp.mean((t - m) * (t - m), axis=0, keepdims=True)
    return (t - m) * lax.rsqrt(v + _EPS)


def _mm(a, b):
    return jnp.dot(a, b, preferred_element_type=jnp.float32,
                   precision=lax.Precision.HIGHEST)


def _tc(body, *args, n_out_shapes):
    return pl.pallas_call(
        body, out_shape=[jax.ShapeDtypeStruct(s, jnp.float32) for s in n_out_shapes]
    )(*args)


def _tc_mm(x, w):
    def body(x_ref, w_ref, o_ref):
        o_ref[...] = _mm(x_ref[...], w_ref[...])
    return _tc(body, x, w, n_out_shapes=[(x.shape[0], w.shape[1])])[0]


def _tc_sum_norm_relu_mm(p, w):
    def body(p_ref, w_ref, h_ref, o_ref):
        h = jnp.maximum(_norm(p_ref[0] + p_ref[1]), 0.0)
        h_ref[...] = h
        o_ref[...] = _mm(h, w_ref[...])
    n = p.shape[1]
    return _tc(body, p, w, n_out_shapes=[(n, p.shape[2]), (n, w.shape[1])])


def _tc_sum_norm_res_relu_mm(p, hprev, wd, wa2):
    def body(p_ref, hp_ref, wd_ref, wa_ref, h_ref, o_ref):
        h = jnp.maximum(_norm(p_ref[0] + p_ref[1]) + _mm(hp_ref[...], wd_ref[...]), 0.0)
        h_ref[...] = h
        o_ref[...] = _mm(h, wa_ref[...])
    n = p.shape[1]
    return _tc(body, p, hprev, wd, wa2,
               n_out_shapes=[(n, p.shape[2]), (n, wa2.shape[1])])


def _tc_sum_norm_resid_relu(p, hprev):
    def body(p_ref, hp_ref, h_ref):
        h_ref[...] = jnp.maximum(_norm(p_ref[0] + p_ref[1]) + hp_ref[...], 0.0)
    return _tc(body, p, hprev, n_out_shapes=[(p.shape[1], p.shape[2])])[0]


def _tc_sum_mm_norm_relu_mm(p, wa, wb):
    def body(p_ref, wa_ref, wb_ref, o_ref):
        t = jnp.maximum(_norm(_mm(p_ref[0] + p_ref[1], wa_ref[...])), 0.0)
        o_ref[...] = _mm(t, wb_ref[...])
    return _tc(body, p, wa, wb, n_out_shapes=[(p.shape[1], wb.shape[1])])[0]


def _tc_final(p, hprev, wf, bf2):
    def body(p_ref, hp_ref, wf_ref, bf_ref, o_ref):
        h = jnp.maximum(_norm(p_ref[0] + p_ref[1]) + hp_ref[...], 0.0)
        o_ref[...] = _mm(h, wf_ref[...]) + bf_ref[...]
    return _tc(body, p, hprev, wf, bf2, n_out_shapes=[(p.shape[1], wf.shape[1])])[0]


def kernel(x, edge_index, W1, l1b1_Wa, l1b1_Wb, l1b1_Wd, l1b2_Wa, l1b2_Wb,
           l2b1_Wa, l2b1_Wb, l2b1_Wd, l2b2_Wa, l2b2_Wb,
           l3b1_Wa, l3b1_Wb, l3b1_Wd, l3b2_Wa, l3b2_Wb, Wf, bf):
    pad = EP - E
    srcr = jnp.concatenate([edge_index[0], jnp.zeros((pad,), jnp.int32)])
    dstr = jnp.concatenate(
        [edge_index[1], N + (jnp.arange(pad, dtype=jnp.int32) % (ACCN - N))])
    bf2 = bf.reshape(1, -1)

    def agg(h):
        return _agg(h, srcr, dstr)

    p = agg(_tc_mm(x, W1))
    h1, o = _tc_sum_norm_relu_mm(p, l1b1_Wa)
    p = agg(o)
    _, o = _tc_sum_norm_relu_mm(p, l1b1_Wb)
    p = agg(o)
    h2, o = _tc_sum_norm_res_relu_mm(p, h1, l1b1_Wd, l1b2_Wa)
    p = agg(o)
    _, o = _tc_sum_norm_relu_mm(p, l1b2_Wb)
    p = agg(o)
    h3 = _tc_sum_norm_resid_relu(p, h2)
    p = agg(h3)
    o = _tc_sum_mm_norm_relu_mm(p, l2b1_Wa, l2b1_Wb)
    p = agg(o)
    h4, o = _tc_sum_norm_res_relu_mm(p, h3, l2b1_Wd, l2b2_Wa)
    p = agg(o)
    _, o = _tc_sum_norm_relu_mm(p, l2b2_Wb)
    p = agg(o)
    h5 = _tc_sum_norm_resid_relu(p, h4)
    p = agg(h5)
    o = _tc_sum_mm_norm_relu_mm(p, l3b1_Wa, l3b1_Wb)
    p = agg(o)
    h6, o = _tc_sum_norm_res_relu_mm(p, h5, l3b1_Wd, l3b2_Wa)
    p = agg(o)
    _, o = _tc_sum_norm_relu_mm(p, l3b2_Wb)
    p = agg(o)
    return _tc_final(p, h6, Wf, bf2)


# speedup vs baseline: 1.4988x; 1.4988x over previous
import functools

import jax
import jax.numpy as jnp
from jax import lax
from jax.experimental import pallas as pl
from jax.experimental.pallas import tpu as pltpu
from jax.experimental.pallas import tpu_sc as plsc

N = 10000
E = 320000
NC = 2
NS = 16
NW = NC * NS
EPW = 10240
EP = NW * EPW
ACCN = 10240
RPT = ACCN // NS
ROUT = 400
ZR = 64

_EPS = 1e-5


@functools.lru_cache(maxsize=None)
def _make_agg(C):
    if C == 128:
        CH, R, G = 64, 4, 2
    else:
        CH, R, G = 128, 4, 2
    STAGE = C <= 64
    K = R - G
    NCHUNK = EPW // CH
    zr = 32
    mesh = plsc.VectorSubcoreMesh(
        core_axis_name="c", subcore_axis_name="s", num_cores=NC, num_subcores=NS
    )

    @functools.partial(
        pl.kernel,
        out_type=jax.ShapeDtypeStruct((NC, N, C), jnp.float32),
        mesh=mesh,
        compiler_params=pltpu.CompilerParams(use_tc_tiling_on_sc=False),
        scratch_types=[
            pltpu.VMEM_SHARED((ACCN, C), jnp.float32),
            pltpu.VMEM_SHARED((N if STAGE else 8, C), jnp.float32),
            [pltpu.VMEM((CH,), jnp.int32) for _ in range(R)],
            [pltpu.VMEM((CH,), jnp.int32) for _ in range(2 * R)],
            [pltpu.VMEM((CH, C), jnp.float32) for _ in range(R)],
            pltpu.VMEM((zr, C), jnp.float32),
            [pltpu.SemaphoreType.DMA for _ in range(R)],
            [pltpu.SemaphoreType.DMA for _ in range(2 * R)],
            [pltpu.SemaphoreType.DMA for _ in range(R)],
            [pltpu.SemaphoreType.DMA for _ in range(R)],
        ],
    )
    def agg(h_hbm, src_hbm, dst_hbm, out_hbm, acc, hs, sbs, dbs, rbs, zb,
            sss, iss, gss, css):
        c = lax.axis_index("c")
        s = lax.axis_index("s")
        wid = c * NS + s

        def zrow(i, _):
            def zcol(k, _):
                zb[i, pl.ds(k * 16, 16)] = jnp.zeros((16,), jnp.float32)
                return 0
            return lax.fori_loop(0, C // 16, zcol, 0)
        lax.fori_loop(0, zr, zrow, 0)
        start = pl.multiple_of(s * RPT, 8)
        for r in range(RPT // zr):
            pltpu.sync_copy(zb, acc.at[pl.ds(start + r * zr, zr)])
        if STAGE:
            @pl.when(s < NS - 1)
            def _():
                pltpu.sync_copy(h_hbm.at[pl.ds(start, RPT)],
                                hs.at[pl.ds(start, RPT)])

            @pl.when(s == NS - 1)
            def _():
                pltpu.sync_copy(h_hbm.at[pl.ds(start, ROUT)],
                                hs.at[pl.ds(start, ROUT)])
        plsc.subcore_barrier()

        def src_cp(j, v):
            off = pl.multiple_of(wid * EPW + j * CH, 8)
            return pltpu.make_async_copy(src_hbm.at[pl.ds(off, CH)],
                                         sbs[v % R], sss[v % R])

        def dst_cp(j, v):
            off = pl.multiple_of(wid * EPW + j * CH, 8)
            return pltpu.make_async_copy(dst_hbm.at[pl.ds(off, CH)],
                                         dbs[v % (2 * R)], iss[v % (2 * R)])

        htab = hs if STAGE else h_hbm

        def row_cp(j, v):
            return pltpu.make_async_copy(htab.at[sbs[v % R]],
                                         rbs[v % R], gss[v % R])

        def add_start(v):
            pltpu.async_copy(rbs[v % R], acc.at[dbs[v % (2 * R)]],
                             css[v % R], add=True)

        def add_wait(v):
            pltpu.make_async_copy(rbs[v % R], acc.at[dbs[v % (2 * R)]],
                                  css[v % R]).wait()

        for q in range(R):
            src_cp(q, q).start()
            dst_cp(q, q).start()
        for q in range(G):
            src_cp(q, q).wait()
            row_cp(q, q).start()

        def step(j, v):
            row_cp(j, v).wait()

            @pl.when(j + R < NCHUNK)
            def _():
                src_cp(j + R, v + R).start()
                dst_cp(j + R, v + R).start()

            dst_cp(j, v).wait()
            add_start(v)

            @pl.when(j >= K)
            def _():
                add_wait(v - K)

            @pl.when(j + G < NCHUNK)
            def _():
                src_cp(j + G, v + G).wait()
                row_cp(j + G, v + G).start()

        def outer(jo, _):
            for v in range(2 * R):
                step(jo * 2 * R + v, v)
            return 0

        lax.fori_loop(0, NCHUNK // (2 * R), outer, 0)

        for k in range(K):
            add_wait(NCHUNK - K + k)
        plsc.subcore_barrier()

        @pl.when(s < NS - 1)
        def _():
            pltpu.sync_copy(acc.at[pl.ds(start, RPT)],
                            out_hbm.at[c, pl.ds(start, RPT)])

        @pl.when(s == NS - 1)
        def _():
            pltpu.sync_copy(acc.at[pl.ds(start, ROUT)],
                            out_hbm.at[c, pl.ds(start, ROUT)])

    return agg


def _agg(h, srcr, dstr):
    return _make_agg(h.shape[1])(h, srcr, dstr)


def _norm(t):
    m = jnp.mean(t, axis=0, keepdims=True)
    v = jnp.mean((t - m) * (t - m), axis=0, keepdims=True)
    return (t - m) * lax.rsqrt(v + _EPS)


def _mm(a, b):
    return jnp.dot(a, b, preferred_element_type=jnp.float32,
                   precision=lax.Precision.HIGHEST)


def _tc(body, *args, n_out_shapes):
    return pl.pallas_call(
        body, out_shape=[jax.ShapeDtypeStruct(s, jnp.float32) for s in n_out_shapes]
    )(*args)


def _tc_mm(x, w):
    def body(x_ref, w_ref, o_ref):
        o_ref[...] = _mm(x_ref[...], w_ref[...])
    return _tc(body, x, w, n_out_shapes=[(x.shape[0], w.shape[1])])[0]


def _tc_sum_norm_relu_mm(p, w):
    def body(p_ref, w_ref, h_ref, o_ref):
        h = jnp.maximum(_norm(p_ref[0] + p_ref[1]), 0.0)
        h_ref[...] = h
        o_ref[...] = _mm(h, w_ref[...])
    n = p.shape[1]
    return _tc(body, p, w, n_out_shapes=[(n, p.shape[2]), (n, w.shape[1])])


def _tc_sum_norm_res_relu_mm(p, hprev, wd, wa2):
    def body(p_ref, hp_ref, wd_ref, wa_ref, h_ref, o_ref):
        h = jnp.maximum(_norm(p_ref[0] + p_ref[1]) + _mm(hp_ref[...], wd_ref[...]), 0.0)
        h_ref[...] = h
        o_ref[...] = _mm(h, wa_ref[...])
    n = p.shape[1]
    return _tc(body, p, hprev, wd, wa2,
               n_out_shapes=[(n, p.shape[2]), (n, wa2.shape[1])])


def _tc_sum_norm_resid_relu(p, hprev):
    def body(p_ref, hp_ref, h_ref):
        h_ref[...] = jnp.maximum(_norm(p_ref[0] + p_ref[1]) + hp_ref[...], 0.0)
    return _tc(body, p, hprev, n_out_shapes=[(p.shape[1], p.shape[2])])[0]


def _tc_sum_mm_norm_relu_mm(p, wa, wb):
    def body(p_ref, wa_ref, wb_ref, o_ref):
        t = jnp.maximum(_norm(_mm(p_ref[0] + p_ref[1], wa_ref[...])), 0.0)
        o_ref[...] = _mm(t, wb_ref[...])
    return _tc(body, p, wa, wb, n_out_shapes=[(p.shape[1], wb.shape[1])])[0]


def _tc_final(p, hprev, wf, bf2):
    def body(p_ref, hp_ref, wf_ref, bf_ref, o_ref):
        h = jnp.maximum(_norm(p_ref[0] + p_ref[1]) + hp_ref[...], 0.0)
        o_ref[...] = _mm(h, wf_ref[...]) + bf_ref[...]
    return _tc(body, p, hprev, wf, bf2, n_out_shapes=[(p.shape[1], wf.shape[1])])[0]


def kernel(x, edge_index, W1, l1b1_Wa, l1b1_Wb, l1b1_Wd, l1b2_Wa, l1b2_Wb,
           l2b1_Wa, l2b1_Wb, l2b1_Wd, l2b2_Wa, l2b2_Wb,
           l3b1_Wa, l3b1_Wb, l3b1_Wd, l3b2_Wa, l3b2_Wb, Wf, bf):
    pad = EP - E
    srcr = jnp.concatenate([edge_index[0], jnp.zeros((pad,), jnp.int32)])
    dstr = jnp.concatenate(
        [edge_index[1], N + (jnp.arange(pad, dtype=jnp.int32) % (ACCN - N))])
    bf2 = bf.reshape(1, -1)

    def agg(h):
        return _agg(h, srcr, dstr)

    p = agg(_tc_mm(x, W1))
    h1, o = _tc_sum_norm_relu_mm(p, l1b1_Wa)
    p = agg(o)
    _, o = _tc_sum_norm_relu_mm(p, l1b1_Wb)
    p = agg(o)
    h2, o = _tc_sum_norm_res_relu_mm(p, h1, l1b1_Wd, l1b2_Wa)
    p = agg(o)
    _, o = _tc_sum_norm_relu_mm(p, l1b2_Wb)
    p = agg(o)
    h3 = _tc_sum_norm_resid_relu(p, h2)
    p = agg(h3)
    o = _tc_sum_mm_norm_relu_mm(p, l2b1_Wa, l2b1_Wb)
    p = agg(o)
    h4, o = _tc_sum_norm_res_relu_mm(p, h3, l2b1_Wd, l2b2_Wa)
    p = agg(o)
    _, o = _tc_sum_norm_relu_mm(p, l2b2_Wb)
    p = agg(o)
    h5 = _tc_sum_norm_resid_relu(p, h4)
    p = agg(h5)
    o = _tc_sum_mm_norm_relu_mm(p, l3b1_Wa, l3b1_Wb)
    p = agg(o)
    h6, o = _tc_sum_norm_res_relu_mm(p, h5, l3b1_Wd, l3b2_Wa)
    p = agg(o)
    _, o = _tc_sum_norm_relu_mm(p, l3b2_Wb)
    p = agg(o)
    return _tc_final(p, h6, Wf, bf2)


# speedup vs baseline: 2.5246x; 1.6844x over previous
import functools

import jax
import jax.numpy as jnp
from jax import lax
from jax.experimental import pallas as pl
from jax.experimental.pallas import tpu as pltpu
from jax.experimental.pallas import tpu_sc as plsc

N = 10000
E = 320000
NC = 2
NS = 16
NW = NC * NS
EPW = 10240
EP = NW * EPW
ACCN = 10240
RPT = ACCN // NS
ROUT = 400
ZR = 64

_EPS = 1e-5


@functools.lru_cache(maxsize=None)
def _make_agg(C):
    if C == 128:
        CH, R, G = 64, 4, 2
    else:
        CH, R, G = 128, 4, 2
    STAGE = C <= 64
    K = R - G
    NCHUNK = EPW // CH
    zr = 32
    mesh = plsc.VectorSubcoreMesh(
        core_axis_name="c", subcore_axis_name="s", num_cores=NC, num_subcores=NS
    )

    @functools.partial(
        pl.kernel,
        out_type=jax.ShapeDtypeStruct((NC, N, C), jnp.float32),
        mesh=mesh,
        compiler_params=pltpu.CompilerParams(use_tc_tiling_on_sc=False),
        scratch_types=[
            pltpu.VMEM_SHARED((ACCN, C), jnp.float32),
            pltpu.VMEM_SHARED((N if STAGE else 8, C), jnp.float32),
            [pltpu.VMEM((CH,), jnp.int32) for _ in range(R)],
            [pltpu.VMEM((CH,), jnp.int32) for _ in range(2 * R)],
            [pltpu.VMEM((CH, C), jnp.float32) for _ in range(R)],
            pltpu.VMEM((zr, C), jnp.float32),
            [pltpu.SemaphoreType.DMA for _ in range(R)],
            [pltpu.SemaphoreType.DMA for _ in range(2 * R)],
            [pltpu.SemaphoreType.DMA for _ in range(R)],
            [pltpu.SemaphoreType.DMA for _ in range(R)],
        ],
    )
    def agg(h_hbm, src_hbm, dst_hbm, out_hbm, acc, hs, sbs, dbs, rbs, zb,
            sss, iss, gss, css):
        c = lax.axis_index("c")
        s = lax.axis_index("s")
        wid = c * NS + s

        def zrow(i, _):
            def zcol(k, _):
                zb[i, pl.ds(k * 16, 16)] = jnp.zeros((16,), jnp.float32)
                return 0
            return lax.fori_loop(0, C // 16, zcol, 0)
        lax.fori_loop(0, zr, zrow, 0)
        start = pl.multiple_of(s * RPT, 8)
        for r in range(RPT // zr):
            pltpu.sync_copy(zb, acc.at[pl.ds(start + r * zr, zr)])
        if STAGE:
            @pl.when(s < NS - 1)
            def _():
                pltpu.sync_copy(h_hbm.at[pl.ds(start, RPT)],
                                hs.at[pl.ds(start, RPT)])

            @pl.when(s == NS - 1)
            def _():
                pltpu.sync_copy(h_hbm.at[pl.ds(start, ROUT)],
                                hs.at[pl.ds(start, ROUT)])
        plsc.subcore_barrier()

        def src_cp(j, v):
            off = pl.multiple_of(wid * EPW + j * CH, 8)
            return pltpu.make_async_copy(src_hbm.at[pl.ds(off, CH)],
                                         sbs[v % R], sss[v % R])

        def dst_cp(j, v):
            off = pl.multiple_of(wid * EPW + j * CH, 8)
            return pltpu.make_async_copy(dst_hbm.at[pl.ds(off, CH)],
                                         dbs[v % (2 * R)], iss[v % (2 * R)])

        htab = hs if STAGE else h_hbm

        def row_cp(j, v):
            return pltpu.make_async_copy(htab.at[sbs[v % R]],
                                         rbs[v % R], gss[v % R])

        def add_start(v):
            pltpu.async_copy(rbs[v % R], acc.at[dbs[v % (2 * R)]],
                             css[v % R], add=True)

        def add_wait(v):
            pltpu.make_async_copy(rbs[v % R], acc.at[dbs[v % (2 * R)]],
                                  css[v % R]).wait()

        for q in range(R):
            src_cp(q, q).start()
            dst_cp(q, q).start()
        for q in range(G):
            src_cp(q, q).wait()
            row_cp(q, q).start()

        def step(j, v):
            row_cp(j, v).wait()

            @pl.when(j + R < NCHUNK)
            def _():
                src_cp(j + R, v + R).start()
                dst_cp(j + R, v + R).start()

            dst_cp(j, v).wait()
            add_start(v)

            @pl.when(j >= K)
            def _():
                add_wait(v - K)

            @pl.when(j + G < NCHUNK)
            def _():
                src_cp(j + G, v + G).wait()
                row_cp(j + G, v + G).start()

        def outer(jo, _):
            for v in range(2 * R):
                step(jo * 2 * R + v, v)
            return 0

        lax.fori_loop(0, NCHUNK // (2 * R), outer, 0)

        for k in range(K):
            add_wait(NCHUNK - K + k)
        plsc.subcore_barrier()

        @pl.when(s < NS - 1)
        def _():
            pltpu.sync_copy(acc.at[pl.ds(start, RPT)],
                            out_hbm.at[c, pl.ds(start, RPT)])

        @pl.when(s == NS - 1)
        def _():
            pltpu.sync_copy(acc.at[pl.ds(start, ROUT)],
                            out_hbm.at[c, pl.ds(start, ROUT)])

    return agg


@functools.lru_cache(maxsize=None)
def _make_agg128():
    C = 64
    CH, R, G = 128, 4, 2
    EPT = EP // NS
    NCHUNK = EPT // CH
    zr = 32
    mesh = plsc.VectorSubcoreMesh(
        core_axis_name="c", subcore_axis_name="s", num_cores=NC, num_subcores=NS
    )

    @functools.partial(
        pl.kernel,
        out_type=jax.ShapeDtypeStruct((NC, N, C), jnp.float32),
        mesh=mesh,
        compiler_params=pltpu.CompilerParams(use_tc_tiling_on_sc=False),
        scratch_types=[
            pltpu.VMEM_SHARED((ACCN, C), jnp.float32),
            pltpu.VMEM_SHARED((N, C), jnp.float32),
            [pltpu.VMEM((CH,), jnp.int32) for _ in range(R)],
            [pltpu.VMEM((CH,), jnp.int32) for _ in range(2 * R)],
            [pltpu.VMEM((CH, C), jnp.float32) for _ in range(R)],
            pltpu.VMEM((zr, C), jnp.float32),
            [pltpu.SemaphoreType.DMA for _ in range(R)],
            [pltpu.SemaphoreType.DMA for _ in range(2 * R)],
            [pltpu.SemaphoreType.DMA for _ in range(R)],
            [pltpu.SemaphoreType.DMA for _ in range(R)],
        ],
    )
    def agg(h_hbm, src_hbm, dst_hbm, out_hbm, acc, hs, sbs, dbs, rbs, zb,
            sss, iss, gss, css):
        c = lax.axis_index("c")
        s = lax.axis_index("s")

        def zrow(i, _):
            def zcol(k, _):
                zb[i, pl.ds(k * 16, 16)] = jnp.zeros((16,), jnp.float32)
                return 0
            return lax.fori_loop(0, C // 16, zcol, 0)
        lax.fori_loop(0, zr, zrow, 0)
        start = pl.multiple_of(s * RPT, 8)
        for r in range(RPT // zr):
            pltpu.sync_copy(zb, acc.at[pl.ds(start + r * zr, zr)])
        col = pl.multiple_of(c * C, 8)

        @pl.when(s < NS - 1)
        def _():
            pltpu.sync_copy(h_hbm.at[pl.ds(start, RPT), pl.ds(col, C)],
                            hs.at[pl.ds(start, RPT)])

        @pl.when(s == NS - 1)
        def _():
            pltpu.sync_copy(h_hbm.at[pl.ds(start, ROUT), pl.ds(col, C)],
                            hs.at[pl.ds(start, ROUT)])
        plsc.subcore_barrier()

        def src_cp(j, v):
            off = pl.multiple_of(s * EPT + j * CH, 8)
            return pltpu.make_async_copy(src_hbm.at[pl.ds(off, CH)],
                                         sbs[v % R], sss[v % R])

        def dst_cp(j, v):
            off = pl.multiple_of(s * EPT + j * CH, 8)
            return pltpu.make_async_copy(dst_hbm.at[pl.ds(off, CH)],
                                         dbs[v % (2 * R)], iss[v % (2 * R)])

        def row_cp(j, v):
            return pltpu.make_async_copy(hs.at[sbs[v % R]],
                                         rbs[v % R], gss[v % R])

        def add_start(v):
            pltpu.async_copy(rbs[v % R], acc.at[dbs[v % (2 * R)]],
                             css[v % R], add=True)

        def add_wait(v):
            pltpu.make_async_copy(rbs[v % R], acc.at[dbs[v % (2 * R)]],
                                  css[v % R]).wait()

        for q in range(R):
            src_cp(q, q).start()
            dst_cp(q, q).start()
        for q in range(G):
            src_cp(q, q).wait()
            row_cp(q, q).start()

        def step(j, v):
            row_cp(j, v).wait()

            @pl.when(j + R < NCHUNK)
            def _():
                src_cp(j + R, v + R).start()
                dst_cp(j + R, v + R).start()

            dst_cp(j, v).wait()
            add_start(v)

            @pl.when(j >= R - G)
            def _():
                add_wait(v - (R - G))

            @pl.when(j + G < NCHUNK)
            def _():
                src_cp(j + G, v + G).wait()
                row_cp(j + G, v + G).start()

        def outer(jo, _):
            for v in range(2 * R):
                step(jo * 2 * R + v, v)
            return 0

        lax.fori_loop(0, NCHUNK // (2 * R), outer, 0)

        for k in range(R - G):
            add_wait(NCHUNK - (R - G) + k)
        plsc.subcore_barrier()

        @pl.when(s < NS - 1)
        def _():
            pltpu.sync_copy(acc.at[pl.ds(start, RPT)],
                            out_hbm.at[c, pl.ds(start, RPT)])

        @pl.when(s == NS - 1)
        def _():
            pltpu.sync_copy(acc.at[pl.ds(start, ROUT)],
                            out_hbm.at[c, pl.ds(start, ROUT)])

    return agg


def _agg(h, srcr, dstr):
    if h.shape[1] == 128:
        return _make_agg128()(h, srcr, dstr)
    return _make_agg(h.shape[1])(h, srcr, dstr)


def _norm(t):
    m = jnp.mean(t, axis=0, keepdims=True)
    v = jnp.mean((t - m) * (t - m), axis=0, keepdims=True)
    return (t - m) * lax.rsqrt(v + _EPS)


def _mm(a, b):
    return jnp.dot(a, b, preferred_element_type=jnp.float32,
                   precision=lax.Precision.HIGHEST)


def _tc(body, *args, n_out_shapes):
    return pl.pallas_call(
        body, out_shape=[jax.ShapeDtypeStruct(s, jnp.float32) for s in n_out_shapes]
    )(*args)


def _tc_mm(x, w):
    def body(x_ref, w_ref, o_ref):
        o_ref[...] = _mm(x_ref[...], w_ref[...])
    return _tc(body, x, w, n_out_shapes=[(x.shape[0], w.shape[1])])[0]


def _comb(p_ref, cat):
    if cat:
        return jnp.concatenate((p_ref[0], p_ref[1]), axis=1)
    return p_ref[0] + p_ref[1]


def _pw(p, cat):
    return p.shape[2] * (2 if cat else 1)


def _tc_sum_norm_relu_mm(p, w, cat=False):
    def body(p_ref, w_ref, h_ref, o_ref):
        h = jnp.maximum(_norm(_comb(p_ref, cat)), 0.0)
        h_ref[...] = h
        o_ref[...] = _mm(h, w_ref[...])
    n = p.shape[1]
    return _tc(body, p, w, n_out_shapes=[(n, _pw(p, cat)), (n, w.shape[1])])


def _tc_sum_norm_res_relu_mm(p, hprev, wd, wa2, cat=False):
    def body(p_ref, hp_ref, wd_ref, wa_ref, h_ref, o_ref):
        h = jnp.maximum(_norm(_comb(p_ref, cat)) + _mm(hp_ref[...], wd_ref[...]), 0.0)
        h_ref[...] = h
        o_ref[...] = _mm(h, wa_ref[...])
    n = p.shape[1]
    return _tc(body, p, hprev, wd, wa2,
               n_out_shapes=[(n, _pw(p, cat)), (n, wa2.shape[1])])


def _tc_sum_norm_resid_relu(p, hprev):
    def body(p_ref, hp_ref, h_ref):
        h_ref[...] = jnp.maximum(_norm(p_ref[0] + p_ref[1]) + hp_ref[...], 0.0)
    return _tc(body, p, hprev, n_out_shapes=[(p.shape[1], p.shape[2])])[0]


def _tc_sum_mm_norm_relu_mm(p, wa, wb):
    def body(p_ref, wa_ref, wb_ref, o_ref):
        t = jnp.maximum(_norm(_mm(p_ref[0] + p_ref[1], wa_ref[...])), 0.0)
        o_ref[...] = _mm(t, wb_ref[...])
    return _tc(body, p, wa, wb, n_out_shapes=[(p.shape[1], wb.shape[1])])[0]


def _tc_final(p, hprev, wf, bf2, cat=False):
    def body(p_ref, hp_ref, wf_ref, bf_ref, o_ref):
        h = jnp.maximum(_norm(_comb(p_ref, cat)) + hp_ref[...], 0.0)
        o_ref[...] = _mm(h, wf_ref[...]) + bf_ref[...]
    return _tc(body, p, hprev, wf, bf2, n_out_shapes=[(p.shape[1], wf.shape[1])])[0]


def kernel(x, edge_index, W1, l1b1_Wa, l1b1_Wb, l1b1_Wd, l1b2_Wa, l1b2_Wb,
           l2b1_Wa, l2b1_Wb, l2b1_Wd, l2b2_Wa, l2b2_Wb,
           l3b1_Wa, l3b1_Wb, l3b1_Wd, l3b2_Wa, l3b2_Wb, Wf, bf):
    pad = EP - E
    srcr = jnp.concatenate([edge_index[0], jnp.zeros((pad,), jnp.int32)])
    dstr = jnp.concatenate(
        [edge_index[1], N + (jnp.arange(pad, dtype=jnp.int32) % (ACCN - N))])
    bf2 = bf.reshape(1, -1)

    def agg(h):
        return _agg(h, srcr, dstr)

    p = agg(_tc_mm(x, W1))
    h1, o = _tc_sum_norm_relu_mm(p, l1b1_Wa)
    p = agg(o)
    _, o = _tc_sum_norm_relu_mm(p, l1b1_Wb)
    p = agg(o)
    h2, o = _tc_sum_norm_res_relu_mm(p, h1, l1b1_Wd, l1b2_Wa)
    p = agg(o)
    _, o = _tc_sum_norm_relu_mm(p, l1b2_Wb)
    p = agg(o)
    h3 = _tc_sum_norm_resid_relu(p, h2)
    p = agg(h3)
    o = _tc_sum_mm_norm_relu_mm(p, l2b1_Wa, l2b1_Wb)
    p = agg(o)
    h4, o = _tc_sum_norm_res_relu_mm(p, h3, l2b1_Wd, l2b2_Wa)
    p = agg(o)
    _, o = _tc_sum_norm_relu_mm(p, l2b2_Wb)
    p = agg(o)
    h5 = _tc_sum_norm_resid_relu(p, h4)
    p = agg(h5)
    o = _tc_sum_mm_norm_relu_mm(p, l3b1_Wa, l3b1_Wb)
    p = agg(o)
    h6, o = _tc_sum_norm_res_relu_mm(p, h5, l3b1_Wd, l3b2_Wa, cat=True)
    p = agg(o)
    _, o = _tc_sum_norm_relu_mm(p, l3b2_Wb, cat=True)
    p = agg(o)
    return _tc_final(p, h6, Wf, bf2, cat=True)
